# Initial kernel scaffold; baseline (speedup 1.0000x reference)
import functools

import jax
import jax.numpy as jnp
from jax import lax
from jax.experimental import pallas as pl
from jax.experimental.pallas import tpu as pltpu
from jax.experimental.pallas import tpu_sc as plsc

_ROWS, _COLS = 16384, 1024
_K = 128
_L = 16
_NC, _NS = 2, 16
_NW = _NC * _NS
_R = 16
_RPW = _ROWS // _NW
_STEPS = _RPW // _R

_mesh = plsc.VectorSubcoreMesh(core_axis_name="c", subcore_axis_name="s")


@functools.partial(
    pl.kernel,
    out_type=jax.ShapeDtypeStruct((_ROWS, _COLS), jnp.float32),
    mesh=_mesh,
    scratch_types=[
        pltpu.VMEM((_R, _COLS), jnp.float32),
        pltpu.VMEM((_R, _K), jnp.int32),
        pltpu.VMEM((_L,), jnp.float32),
    ],
)
def _scatter_rows(x_hbm, idx_hbm, val_hbm, out_hbm, xbuf, ibuf, vbuf):
    wid = lax.axis_index("s") * _NC + lax.axis_index("c")
    pltpu.sync_copy(val_hbm, vbuf)
    val = vbuf[...]
    base = wid * _RPW

    def step(s, carry):
        row0 = base + s * _R
        pltpu.sync_copy(x_hbm.at[pl.ds(row0, _R)], xbuf)
        pltpu.sync_copy(idx_hbm.at[pl.ds(row0, _R)], ibuf)
        for r in range(_R):
            rowv = jnp.full((_L,), r, jnp.int32)
            for j in range(_K // _L):
                colv = ibuf[r, pl.ds(j * _L, _L)]
                plsc.store_scatter(xbuf, [rowv, colv], val)
        pltpu.sync_copy(xbuf, out_hbm.at[pl.ds(row0, _R)])
        return carry

    lax.fori_loop(0, _STEPS, step, 0)


def kernel(x, dim, index, value, out):
    del dim, out
    vals = jnp.broadcast_to(jnp.asarray(value, jnp.float32), (_L,))
    return _scatter_rows(x, index, vals)


# speedup vs baseline: 37.3743x; 37.3743x over previous
import functools

import jax
import jax.numpy as jnp
from jax import lax
from jax.experimental import pallas as pl
from jax.experimental.pallas import tpu as pltpu
from jax.experimental.pallas import tpu_sc as plsc

_ROWS, _COLS = 16384, 1024
_K = 128
_L = 16
_NC, _NS = 2, 16
_NW = _NC * _NS
_R = 16
_RPW = _ROWS // _NW
_STEPS = _RPW // _R

_mesh = plsc.VectorSubcoreMesh(core_axis_name="c", subcore_axis_name="s")


@functools.partial(
    pl.kernel,
    out_type=jax.ShapeDtypeStruct((_ROWS * _COLS,), jnp.float32),
    mesh=_mesh,
    compiler_params=pltpu.CompilerParams(needs_layout_passes=False),
    scratch_types=[
        pltpu.VMEM((_R * _COLS,), jnp.float32),
        pltpu.VMEM((_R * _K,), jnp.int32),
        pltpu.VMEM((_L,), jnp.float32),
    ],
)
def _scatter_rows(x_hbm, idx_hbm, val_hbm, out_hbm, xbuf, ibuf, vbuf):
    wid = lax.axis_index("s") * _NC + lax.axis_index("c")
    pltpu.sync_copy(val_hbm, vbuf)
    val = vbuf[...]
    base = wid * _RPW

    def step(s, carry):
        row0 = base + s * _R
        pltpu.sync_copy(x_hbm.at[pl.ds(row0 * _COLS, _R * _COLS)], xbuf)
        pltpu.sync_copy(idx_hbm.at[pl.ds(row0 * _K, _R * _K)], ibuf)
        for r in range(_R):
            for j in range(_K // _L):
                colv = ibuf[pl.ds(r * _K + j * _L, _L)]
                plsc.store_scatter(xbuf, [colv + r * _COLS], val)
        pltpu.sync_copy(xbuf, out_hbm.at[pl.ds(row0 * _COLS, _R * _COLS)])
        return carry

    lax.fori_loop(0, _STEPS, step, 0)


def kernel(x, dim, index, value, out):
    del dim, out
    vals = jnp.broadcast_to(jnp.asarray(value, jnp.float32), (_L,))
    res = _scatter_rows(x.reshape(-1), index.reshape(-1), vals)
    return res.reshape(_ROWS, _COLS)


# speedup vs baseline: 49.3887x; 1.3215x over previous
import functools

import jax
import jax.numpy as jnp
from jax import lax
from jax.experimental import pallas as pl
from jax.experimental.pallas import tpu as pltpu
from jax.experimental.pallas import tpu_sc as plsc

_ROWS, _COLS = 16384, 1024
_K = 128
_L = 16
_NC, _NS = 2, 16
_NW = _NC * _NS
_R = 16
_RPW = _ROWS // _NW
_S = _RPW // _R
_NBUF = 4
_CW = _R * _COLS
_IW = _R * _K

_mesh = plsc.VectorSubcoreMesh(core_axis_name="c", subcore_axis_name="s")


@functools.partial(
    pl.kernel,
    out_type=jax.ShapeDtypeStruct((_ROWS * _COLS,), jnp.float32),
    mesh=_mesh,
    compiler_params=pltpu.CompilerParams(needs_layout_passes=False),
    scratch_types=[
        [pltpu.VMEM((_CW,), jnp.float32) for _ in range(_NBUF)],
        [pltpu.VMEM((_IW,), jnp.int32) for _ in range(_NBUF)],
        pltpu.VMEM((_L,), jnp.float32),
        [pltpu.SemaphoreType.DMA for _ in range(_NBUF)],
        [pltpu.SemaphoreType.DMA for _ in range(_NBUF)],
    ],
)
def _scatter_rows(x_hbm, idx_hbm, val_hbm, out_hbm, xbufs, ibufs, vbuf,
                  ldsems, stsems):
    wid = lax.axis_index("s") * _NC + lax.axis_index("c")
    pltpu.sync_copy(val_hbm, vbuf)
    val = vbuf[...]
    cbase = wid * _S

    def load(c, b):
        pltpu.async_copy(x_hbm.at[pl.ds((cbase + c) * _CW, _CW)], xbufs[b],
                         ldsems[b])
        pltpu.async_copy(idx_hbm.at[pl.ds((cbase + c) * _IW, _IW)], ibufs[b],
                         ldsems[b])

    def wait_load(c, b):
        pltpu.make_async_copy(x_hbm.at[pl.ds((cbase + c) * _CW, _CW)],
                              xbufs[b], ldsems[b]).wait()
        pltpu.make_async_copy(idx_hbm.at[pl.ds((cbase + c) * _IW, _IW)],
                              ibufs[b], ldsems[b]).wait()

    def store(c, b):
        pltpu.async_copy(xbufs[b], out_hbm.at[pl.ds((cbase + c) * _CW, _CW)],
                         stsems[b])

    def wait_store(c, b):
        pltpu.make_async_copy(xbufs[b], out_hbm.at[pl.ds((cbase + c) * _CW, _CW)],
                              stsems[b]).wait()

    def scatter(b):
        for r in range(_R):
            row = xbufs[b].at[pl.ds(r * _COLS, _COLS)]
            for j in range(_K // _L):
                colv = ibufs[b][pl.ds(r * _K + j * _L, _L)]
                plsc.store_scatter(row, [colv], val)

    for b in range(_NBUF):
        load(b, b)

    def outer(o, carry):
        for b in range(_NBUF):
            s = o * _NBUF + b
            br = (b - 2) % _NBUF

            @pl.when((s >= 2) & (s + _NBUF - 2 < _S))
            def _():
                wait_store(s - 2, br)
                load(s + _NBUF - 2, br)

            wait_load(s, b)
            scatter(b)
            store(s, b)
        return carry

    lax.fori_loop(0, _S // _NBUF, outer, 0)

    for i in range(_NBUF):
        c = _S - _NBUF + i
        wait_store(c, c % _NBUF)


def kernel(x, dim, index, value, out):
    del dim, out
    vals = jnp.broadcast_to(jnp.asarray(value, jnp.float32), (_L,))
    res = _scatter_rows(x.reshape(-1), index.reshape(-1), vals)
    return res.reshape(_ROWS, _COLS)


# speedup vs baseline: 112.9164x; 2.2863x over previous
import functools

import jax
import jax.numpy as jnp
from jax import lax
from jax.experimental import pallas as pl
from jax.experimental.pallas import tpu as pltpu
from jax.experimental.pallas import tpu_sc as plsc

_ROWS, _COLS = 16384, 1024
_K = 128
_L = 16
_NC, _NS = 2, 16
_NW = _NC * _NS
_R = 16
_RPW = _ROWS // _NW
_S = _RPW // _R
_NBUF = 4

_mesh = plsc.VectorSubcoreMesh(core_axis_name="c", subcore_axis_name="s")


@functools.partial(
    pl.kernel,
    out_type=jax.ShapeDtypeStruct((_ROWS, _COLS), jnp.float32),
    mesh=_mesh,
    compiler_params=pltpu.CompilerParams(needs_layout_passes=False),
    scratch_types=[
        [pltpu.VMEM((_R, _COLS), jnp.float32) for _ in range(_NBUF)],
        [pltpu.VMEM((_R, _K), jnp.int32) for _ in range(_NBUF)],
        pltpu.VMEM((_L,), jnp.float32),
        [pltpu.SemaphoreType.DMA for _ in range(_NBUF)],
        [pltpu.SemaphoreType.DMA for _ in range(_NBUF)],
    ],
)
def _scatter_rows(x_hbm, idx_hbm, val_hbm, out_hbm, xbufs, ibufs, vbuf,
                  ldsems, stsems):
    wid = lax.axis_index("s") * _NC + lax.axis_index("c")
    pltpu.sync_copy(val_hbm, vbuf)
    val = vbuf[...]
    cbase = wid * _S

    def load(c, b):
        r0 = (cbase + c) * _R
        pltpu.async_copy(x_hbm.at[pl.ds(r0, _R)], xbufs[b], ldsems[b])
        pltpu.async_copy(idx_hbm.at[pl.ds(r0, _R)], ibufs[b], ldsems[b])

    def wait_load(c, b):
        r0 = (cbase + c) * _R
        pltpu.make_async_copy(x_hbm.at[pl.ds(r0, _R)], xbufs[b],
                              ldsems[b]).wait()
        pltpu.make_async_copy(idx_hbm.at[pl.ds(r0, _R)], ibufs[b],
                              ldsems[b]).wait()

    def store(c, b):
        r0 = (cbase + c) * _R
        pltpu.async_copy(xbufs[b], out_hbm.at[pl.ds(r0, _R)], stsems[b])

    def wait_store(c, b):
        r0 = (cbase + c) * _R
        pltpu.make_async_copy(xbufs[b], out_hbm.at[pl.ds(r0, _R)],
                              stsems[b]).wait()

    def scatter(b):
        for r in range(_R):
            rowv = jnp.full((_L,), r, jnp.int32)
            for j in range(_K // _L):
                colv = ibufs[b][r, pl.ds(j * _L, _L)]
                plsc.store_scatter(xbufs[b], [rowv, colv], val)

    for b in range(_NBUF):
        load(b, b)

    def outer(o, carry):
        for b in range(_NBUF):
            s = o * _NBUF + b
            br = (b - 2) % _NBUF

            @pl.when((s >= 2) & (s + _NBUF - 2 < _S))
            def _():
                wait_store(s - 2, br)
                load(s + _NBUF - 2, br)

            wait_load(s, b)
            scatter(b)
            store(s, b)
        return carry

    lax.fori_loop(0, _S // _NBUF, outer, 0)

    for i in range(_NBUF):
        c = _S - _NBUF + i
        wait_store(c, c % _NBUF)


def kernel(x, dim, index, value, out):
    del dim, out
    vals = jnp.broadcast_to(jnp.asarray(value, jnp.float32), (_L,))
    return _scatter_rows(x, index, vals)


# speedup vs baseline: 137.9610x; 1.2218x over previous
import functools

import jax
import jax.numpy as jnp
from jax import lax
from jax.experimental import pallas as pl
from jax.experimental.pallas import tpu as pltpu
from jax.experimental.pallas import tpu_sc as plsc

_ROWS, _COLS = 16384, 1024
_K = 128
_L = 16
_NC, _NS = 2, 16
_NW = _NC * _NS
_R = 32
_RPW = _ROWS // _NW
_S = _RPW // _R
_NBUF = 2
_LAG = 1

_mesh = plsc.VectorSubcoreMesh(core_axis_name="c", subcore_axis_name="s")


@functools.partial(
    pl.kernel,
    out_type=jax.ShapeDtypeStruct((_ROWS, _COLS), jnp.float32),
    mesh=_mesh,
    compiler_params=pltpu.CompilerParams(needs_layout_passes=False),
    scratch_types=[
        [pltpu.VMEM((_R, _COLS), jnp.float32) for _ in range(_NBUF)],
        [pltpu.VMEM((_R, _K), jnp.int32) for _ in range(_NBUF)],
        pltpu.VMEM((_L,), jnp.float32),
        [pltpu.SemaphoreType.DMA for _ in range(_NBUF)],
        [pltpu.SemaphoreType.DMA for _ in range(_NBUF)],
    ],
)
def _scatter_rows(x_hbm, idx_hbm, val_hbm, out_hbm, xbufs, ibufs, vbuf,
                  ldsems, stsems):
    wid = lax.axis_index("s") * _NC + lax.axis_index("c")
    cbase = wid * _S

    def load(c, b):
        r0 = (cbase + c) * _R
        pltpu.async_copy(x_hbm.at[pl.ds(r0, _R)], xbufs[b], ldsems[b])
        pltpu.async_copy(idx_hbm.at[pl.ds(r0, _R)], ibufs[b], ldsems[b])

    def wait_load(c, b):
        r0 = (cbase + c) * _R
        pltpu.make_async_copy(x_hbm.at[pl.ds(r0, _R)], xbufs[b],
                              ldsems[b]).wait()
        pltpu.make_async_copy(idx_hbm.at[pl.ds(r0, _R)], ibufs[b],
                              ldsems[b]).wait()

    def store(c, b):
        r0 = (cbase + c) * _R
        pltpu.async_copy(xbufs[b], out_hbm.at[pl.ds(r0, _R)], stsems[b])

    def wait_store(c, b):
        r0 = (cbase + c) * _R
        pltpu.make_async_copy(xbufs[b], out_hbm.at[pl.ds(r0, _R)],
                              stsems[b]).wait()

    def scatter(b):
        @plsc.parallel_loop(0, _R * (_K // _L), unroll=8)
        def _(p):
            r = p >> 3
            j = p & 7
            rowv = jnp.broadcast_to(r, (_L,))
            colv = ibufs[b][r, pl.ds(j * _L, _L)]
            plsc.store_scatter(xbufs[b], [rowv, colv], val)

    for b in range(_NBUF):
        load(b, b)
    pltpu.sync_copy(val_hbm, vbuf)
    val = vbuf[...]

    def outer(o, carry):
        for b in range(_NBUF):
            s = o * _NBUF + b
            br = (b - _LAG) % _NBUF

            @pl.when((s >= _LAG) & (s + _NBUF - _LAG < _S))
            def _():
                wait_store(s - _LAG, br)
                load(s + _NBUF - _LAG, br)

            wait_load(s, b)
            scatter(b)
            store(s, b)
        return carry

    lax.fori_loop(0, _S // _NBUF, outer, 0)

    for i in range(_NBUF):
        c = _S - _NBUF + i
        wait_store(c, c % _NBUF)


def kernel(x, dim, index, value, out):
    del dim, out
    vals = jnp.broadcast_to(jnp.asarray(value, jnp.float32), (_L,))
    return _scatter_rows(x, index, vals)
